# baseline (speedup 1.0000x reference)
import functools

import jax
import jax.numpy as jnp
from jax import lax
from jax.experimental import pallas as pl
from jax.experimental.pallas import tpu as pltpu
from jax.experimental.pallas import tpu_sc as plsc

EMBED_DIM = 32
LANES = 16


def kernel(user_id, movie_id, user_table, movie_table, W, b):
    B = user_id.shape[0]
    info = plsc.get_sparse_core_info()
    nw = info.num_cores * info.num_subcores
    bw = B // nw
    ngroups = bw // LANES

    wb = jnp.zeros((80,), jnp.float32).at[:2 * EMBED_DIM].set(W[:, 0]).at[2 * EMBED_DIM].set(b[0])

    mesh = plsc.VectorSubcoreMesh(core_axis_name="c", subcore_axis_name="s")

    @functools.partial(
        pl.kernel,
        mesh=mesh,
        compiler_params=pltpu.CompilerParams(
            needs_layout_passes=False, use_tc_tiling_on_sc=False),
        out_type=jax.ShapeDtypeStruct((B,), jnp.float32),
        scratch_types=[
            pltpu.VMEM((bw,), jnp.int32),
            pltpu.VMEM((bw,), jnp.int32),
            pltpu.VMEM((bw, EMBED_DIM), jnp.float32),
            pltpu.VMEM((bw, EMBED_DIM), jnp.float32),
            pltpu.VMEM((80,), jnp.float32),
            pltpu.VMEM((bw,), jnp.float32),
            pltpu.SemaphoreType.DMA,
            pltpu.SemaphoreType.DMA,
        ],
    )
    def sc_kernel(uid_hbm, mid_hbm, ut_hbm, mt_hbm, wb_hbm, out_hbm,
                  uidx_v, midx_v, urows_v, mrows_v, w_v, out_v, sem_u, sem_m):
        wid = lax.axis_index("s") * info.num_cores + lax.axis_index("c")
        base = wid * bw
        pltpu.sync_copy(uid_hbm.at[pl.ds(base, bw)], uidx_v)
        pltpu.sync_copy(mid_hbm.at[pl.ds(base, bw)], midx_v)
        cu = pltpu.async_copy(ut_hbm.at[uidx_v], urows_v, sem_u)
        cm = pltpu.async_copy(mt_hbm.at[midx_v], mrows_v, sem_m)
        pltpu.sync_copy(wb_hbm, w_v)
        w0 = w_v[pl.ds(0, LANES)]
        w1 = w_v[pl.ds(LANES, LANES)]
        w2 = w_v[pl.ds(2 * LANES, LANES)]
        w3 = w_v[pl.ds(3 * LANES, LANES)]
        bias = w_v[pl.ds(4 * LANES, LANES)][0]
        wu = [w0[i] for i in range(LANES)] + [w1[i] for i in range(LANES)]
        wm = [w2[i] for i in range(LANES)] + [w3[i] for i in range(LANES)]
        cu.wait()
        cm.wait()

        iota16 = lax.iota(jnp.int32, LANES)

        def group(g, carry):
            rows = g * LANES + iota16
            acc = jnp.zeros((LANES,), jnp.float32) + bias
            for d in range(EMBED_DIM):
                col = jnp.full((LANES,), d, jnp.int32)
                acc = acc + plsc.load_gather(urows_v, [rows, col]) * wu[d]
                acc = acc + plsc.load_gather(mrows_v, [rows, col]) * wm[d]
            out_v[pl.ds(g * LANES, LANES)] = acc
            return carry

        lax.fori_loop(0, ngroups, group, 0)
        pltpu.sync_copy(out_v, out_hbm.at[pl.ds(base, bw)])

    out = sc_kernel(user_id, movie_id, user_table, movie_table, wb)
    return out.reshape(B, 1)


# speedup vs baseline: 7.9325x; 7.9325x over previous
import functools

import jax
import jax.numpy as jnp
from jax import lax
from jax.experimental import pallas as pl
from jax.experimental.pallas import tpu as pltpu
from jax.experimental.pallas import tpu_sc as plsc

EMBED_DIM = 32
LANES = 16
NROWS = 1000000
CHUNK = 16384


def _matvec_body(ut_ref, mt_ref, w_ref, p_ref, q_ref):
    p = jnp.full((CHUNK,), w_ref[2 * EMBED_DIM], jnp.float32)
    q = jnp.zeros((CHUNK,), jnp.float32)
    for d in range(EMBED_DIM):
        p = p + ut_ref[d] * w_ref[d]
        q = q + mt_ref[d] * w_ref[EMBED_DIM + d]
    p_ref[...] = p
    q_ref[...] = q


def _combine(user_id, movie_id, p, q):
    B = user_id.shape[0]
    info = plsc.get_sparse_core_info()
    nw = info.num_cores * info.num_subcores
    bw = B // nw

    mesh = plsc.VectorSubcoreMesh(core_axis_name="c", subcore_axis_name="s")

    @functools.partial(
        pl.kernel,
        mesh=mesh,
        compiler_params=pltpu.CompilerParams(
            needs_layout_passes=False, use_tc_tiling_on_sc=False),
        out_type=jax.ShapeDtypeStruct((B,), jnp.float32),
        scratch_types=[
            pltpu.VMEM((bw,), jnp.int32),
            pltpu.VMEM((bw,), jnp.int32),
            pltpu.VMEM((bw,), jnp.float32),
            pltpu.VMEM((bw,), jnp.float32),
            pltpu.VMEM((bw,), jnp.float32),
            pltpu.SemaphoreType.DMA,
            pltpu.SemaphoreType.DMA,
        ],
    )
    def sc_kernel(uid_hbm, mid_hbm, p_hbm, q_hbm, out_hbm,
                  uidx_v, midx_v, pv, qv, out_v, sem_p, sem_q):
        wid = lax.axis_index("s") * info.num_cores + lax.axis_index("c")
        base = wid * bw
        pltpu.sync_copy(uid_hbm.at[pl.ds(base, bw)], uidx_v)
        pltpu.sync_copy(mid_hbm.at[pl.ds(base, bw)], midx_v)
        cp = pltpu.async_copy(p_hbm.at[uidx_v], pv, sem_p)
        cq = pltpu.async_copy(q_hbm.at[midx_v], qv, sem_q)
        cp.wait()
        cq.wait()
        for g in range(bw // LANES):
            sl = pl.ds(g * LANES, LANES)
            out_v[sl] = pv[sl] + qv[sl]
        pltpu.sync_copy(out_v, out_hbm.at[pl.ds(base, bw)])

    return sc_kernel(user_id, movie_id, p, q)


def kernel(user_id, movie_id, user_table, movie_table, W, b):
    B = user_id.shape[0]
    utT = user_table.T
    mtT = movie_table.T
    wb = jnp.zeros((80,), jnp.float32).at[:2 * EMBED_DIM].set(W[:, 0]).at[2 * EMBED_DIM].set(b[0])

    grid = (NROWS + CHUNK - 1) // CHUNK
    p, q = pl.pallas_call(
        _matvec_body,
        grid=(grid,),
        in_specs=[
            pl.BlockSpec((EMBED_DIM, CHUNK), lambda g: (0, g)),
            pl.BlockSpec((EMBED_DIM, CHUNK), lambda g: (0, g)),
            pl.BlockSpec(memory_space=pltpu.SMEM),
        ],
        out_specs=[
            pl.BlockSpec((CHUNK,), lambda g: (g,)),
            pl.BlockSpec((CHUNK,), lambda g: (g,)),
        ],
        out_shape=[
            jax.ShapeDtypeStruct((NROWS,), jnp.float32),
            jax.ShapeDtypeStruct((NROWS,), jnp.float32),
        ],
        compiler_params=pltpu.CompilerParams(
            dimension_semantics=("arbitrary",)),
    )(utT, mtT, wb)

    out = _combine(user_id, movie_id, p, q)
    return out.reshape(B, 1)


# speedup vs baseline: 9.0552x; 1.1415x over previous
import functools

import jax
import jax.numpy as jnp
from jax import lax
from jax.experimental import pallas as pl
from jax.experimental.pallas import tpu as pltpu
from jax.experimental.pallas import tpu_sc as plsc

EMBED_DIM = 32
LANES = 16
NROWS = 1000000
CHUNK = 32768


def _matvec_body(ut_ref, mt_ref, w_ref, p_ref, q_ref):
    p = jnp.full((CHUNK,), w_ref[2 * EMBED_DIM], jnp.float32)
    q = jnp.zeros((CHUNK,), jnp.float32)
    for d in range(EMBED_DIM):
        p = p + ut_ref[d] * w_ref[d]
        q = q + mt_ref[d] * w_ref[EMBED_DIM + d]
    p_ref[...] = p
    q_ref[...] = q


def _combine(user_id, movie_id, p, q):
    B = user_id.shape[0]
    info = plsc.get_sparse_core_info()
    nw = info.num_cores * info.num_subcores
    bw = B // nw

    mesh = plsc.VectorSubcoreMesh(core_axis_name="c", subcore_axis_name="s")

    @functools.partial(
        pl.kernel,
        mesh=mesh,
        compiler_params=pltpu.CompilerParams(
            needs_layout_passes=False, use_tc_tiling_on_sc=False),
        out_type=jax.ShapeDtypeStruct((B,), jnp.float32),
        scratch_types=[
            pltpu.VMEM((bw,), jnp.int32),
            pltpu.VMEM((bw,), jnp.int32),
            pltpu.VMEM((bw,), jnp.float32),
            pltpu.VMEM((bw,), jnp.float32),
            pltpu.VMEM((bw,), jnp.float32),
            pltpu.SemaphoreType.DMA,
            pltpu.SemaphoreType.DMA,
        ],
    )
    def sc_kernel(uid_hbm, mid_hbm, p_hbm, q_hbm, out_hbm,
                  uidx_v, midx_v, pv, qv, out_v, sem_p, sem_q):
        wid = lax.axis_index("s") * info.num_cores + lax.axis_index("c")
        base = wid * bw
        pltpu.sync_copy(uid_hbm.at[pl.ds(base, bw)], uidx_v)
        pltpu.sync_copy(mid_hbm.at[pl.ds(base, bw)], midx_v)
        cp = pltpu.async_copy(p_hbm.at[uidx_v], pv, sem_p)
        cq = pltpu.async_copy(q_hbm.at[midx_v], qv, sem_q)
        cp.wait()
        cq.wait()
        for g in range(bw // LANES):
            sl = pl.ds(g * LANES, LANES)
            out_v[sl] = pv[sl] + qv[sl]
        pltpu.sync_copy(out_v, out_hbm.at[pl.ds(base, bw)])

    return sc_kernel(user_id, movie_id, p, q)


def kernel(user_id, movie_id, user_table, movie_table, W, b):
    B = user_id.shape[0]
    utT = user_table.T
    mtT = movie_table.T
    wb = jnp.zeros((80,), jnp.float32).at[:2 * EMBED_DIM].set(W[:, 0]).at[2 * EMBED_DIM].set(b[0])

    grid = (NROWS + CHUNK - 1) // CHUNK
    p, q = pl.pallas_call(
        _matvec_body,
        grid=(grid,),
        in_specs=[
            pl.BlockSpec((EMBED_DIM, CHUNK), lambda g: (0, g)),
            pl.BlockSpec((EMBED_DIM, CHUNK), lambda g: (0, g)),
            pl.BlockSpec(memory_space=pltpu.SMEM),
        ],
        out_specs=[
            pl.BlockSpec((CHUNK,), lambda g: (g,)),
            pl.BlockSpec((CHUNK,), lambda g: (g,)),
        ],
        out_shape=[
            jax.ShapeDtypeStruct((NROWS,), jnp.float32),
            jax.ShapeDtypeStruct((NROWS,), jnp.float32),
        ],
        compiler_params=pltpu.CompilerParams(
            dimension_semantics=("arbitrary",)),
    )(utT, mtT, wb)

    out = _combine(user_id, movie_id, p, q)
    return out.reshape(B, 1)
